# Initial kernel scaffold; baseline (speedup 1.0000x reference)
import jax
import jax.numpy as jnp
from jax.experimental import pallas as pl


def kernel(x, edge_index, batch, W1, b1, W2, b2, Wl, bl):
    raise NotImplementedError("write your pallas kernel here")


# speedup vs baseline: 19.3029x; 19.3029x over previous
import functools

import jax
import jax.numpy as jnp
from jax import lax
from jax.experimental import pallas as pl
from jax.experimental.pallas import tpu as pltpu
from jax.experimental.pallas import tpu_sc as plsc

N_NODES = 10000
N_EDGES = 320000
DIM = 128
NUM_GRAPHS = 64

NC = 2
NS = 16
NW = NC * NS
EDGES_PER_TILE = N_EDGES // NW
CHUNK = 125
NCHUNK = EDGES_PER_TILE // CHUNK
RPT = 624
TAIL_BASE = NS * RPT
TAIL = N_NODES - TAIL_BASE

_mesh = plsc.VectorSubcoreMesh(core_axis_name="c", subcore_axis_name="s")


DEGW = 128


@functools.partial(
    pl.kernel,
    mesh=_mesh,
    out_type=jax.ShapeDtypeStruct((NC, N_NODES, DEGW), jnp.float32),
    scratch_types=[
        pltpu.VMEM((NCHUNK, CHUNK), jnp.int32),
        pltpu.VMEM((CHUNK, DEGW), jnp.float32),
        pltpu.VMEM_SHARED((N_NODES, DEGW), jnp.float32),
    ],
)
def _deg_kernel(dst_hbm, ones_hbm, zeros_hbm, out_hbm, dst_v, ones_v, deg_sh):
    cid = lax.axis_index("c")
    sid = lax.axis_index("s")
    wid = cid * NS + sid
    rbase = sid * RPT

    pltpu.sync_copy(dst_hbm.at[pl.ds(wid * NCHUNK, NCHUNK)], dst_v)
    pltpu.sync_copy(ones_hbm, ones_v)
    pltpu.sync_copy(zeros_hbm.at[pl.ds(rbase, RPT)],
                    deg_sh.at[pl.ds(rbase, RPT)])

    @pl.when(sid == NS - 1)
    def _():
        pltpu.sync_copy(zeros_hbm.at[pl.ds(TAIL_BASE, TAIL)],
                        deg_sh.at[pl.ds(TAIL_BASE, TAIL)])

    plsc.subcore_barrier()

    def body(j, carry):
        pltpu.sync_copy(ones_v, deg_sh.at[dst_v.at[j]], add=True)
        return carry

    lax.fori_loop(0, NCHUNK, body, 0)
    plsc.subcore_barrier()
    pltpu.sync_copy(deg_sh.at[pl.ds(rbase, RPT)],
                    out_hbm.at[cid].at[pl.ds(rbase, RPT)])

    @pl.when(sid == NS - 1)
    def _():
        pltpu.sync_copy(deg_sh.at[pl.ds(TAIL_BASE, TAIL)],
                        out_hbm.at[cid].at[pl.ds(TAIL_BASE, TAIL)])


_DBG_LINEAR_GATHER = False


@functools.partial(
    pl.kernel,
    mesh=_mesh,
    out_type=jax.ShapeDtypeStruct((NC, N_NODES, DIM), jnp.float32),
    scratch_types=[
        pltpu.VMEM((NCHUNK, CHUNK), jnp.int32),
        pltpu.VMEM((NCHUNK, CHUNK), jnp.int32),
        pltpu.VMEM((CHUNK, DIM), jnp.float32),
        pltpu.VMEM_SHARED((N_NODES, DIM), jnp.float32),
        pltpu.SemaphoreType.DMA,
    ],
)
def _agg_kernel(y_hbm, src_hbm, dst_hbm, zeros_hbm, out_hbm,
                src_v, dst_v, rows_v, acc_sh, sem):
    cid = lax.axis_index("c")
    sid = lax.axis_index("s")
    wid = cid * NS + sid
    rbase = sid * RPT

    if not _DBG_LINEAR_GATHER:
        pltpu.sync_copy(src_hbm.at[pl.ds(wid * NCHUNK, NCHUNK)], src_v)
    pltpu.sync_copy(dst_hbm.at[pl.ds(wid * NCHUNK, NCHUNK)], dst_v)

    @pl.when(cid == 0)
    def _():
        pltpu.sync_copy(y_hbm.at[pl.ds(rbase, RPT)],
                        acc_sh.at[pl.ds(rbase, RPT)])

        @pl.when(sid == NS - 1)
        def _():
            pltpu.sync_copy(y_hbm.at[pl.ds(TAIL_BASE, TAIL)],
                            acc_sh.at[pl.ds(TAIL_BASE, TAIL)])

    @pl.when(cid == 1)
    def _():
        pltpu.sync_copy(zeros_hbm.at[pl.ds(rbase, RPT)],
                        acc_sh.at[pl.ds(rbase, RPT)])

        @pl.when(sid == NS - 1)
        def _():
            pltpu.sync_copy(zeros_hbm.at[pl.ds(TAIL_BASE, TAIL)],
                            acc_sh.at[pl.ds(TAIL_BASE, TAIL)])

    plsc.subcore_barrier()

    def body(j, carry):
        if _DBG_LINEAR_GATHER:
            pltpu.sync_copy(src_hbm.at[wid * NCHUNK + j], rows_v)
        else:
            pltpu.async_copy(y_hbm.at[src_v.at[j]], rows_v, sem).wait()
        pltpu.sync_copy(rows_v, acc_sh.at[dst_v.at[j]], add=True)
        return carry

    lax.fori_loop(0, NCHUNK, body, 0)
    plsc.subcore_barrier()
    pltpu.sync_copy(acc_sh.at[pl.ds(rbase, RPT)],
                    out_hbm.at[cid].at[pl.ds(rbase, RPT)])

    @pl.when(sid == NS - 1)
    def _():
        pltpu.sync_copy(acc_sh.at[pl.ds(TAIL_BASE, TAIL)],
                        out_hbm.at[cid].at[pl.ds(TAIL_BASE, TAIL)])


_RB = 1000
_GRID = N_NODES // _RB

_tc_params = pltpu.CompilerParams(dimension_semantics=("arbitrary",))


def _tc_a_body(x_ref, w_ref, da_ref, db_ref, y_ref, dis_ref):
    d = da_ref[:, :1] + db_ref[:, :1] + 1.0
    dis = lax.rsqrt(d)
    xw = jnp.dot(x_ref[...], w_ref[...], preferred_element_type=jnp.float32)
    y_ref[...] = xw * dis
    dis_ref[...] = jnp.broadcast_to(dis, (_RB, 8))


def _tc_a(x, w1, deg_a, deg_b):
    return pl.pallas_call(
        _tc_a_body,
        grid=(_GRID,),
        in_specs=[
            pl.BlockSpec((_RB, DIM), lambda i: (i, 0)),
            pl.BlockSpec((DIM, DIM), lambda i: (0, 0)),
            pl.BlockSpec((_RB, DEGW), lambda i: (i, 0)),
            pl.BlockSpec((_RB, DEGW), lambda i: (i, 0)),
        ],
        out_specs=[
            pl.BlockSpec((_RB, DIM), lambda i: (i, 0)),
            pl.BlockSpec((_RB, 8), lambda i: (i, 0)),
        ],
        out_shape=[
            jax.ShapeDtypeStruct((N_NODES, DIM), jnp.float32),
            jax.ShapeDtypeStruct((N_NODES, 8), jnp.float32),
        ],
        compiler_params=_tc_params,
    )(x, w1, deg_a, deg_b)


def _tc_b_body(a0_ref, a1_ref, dis_ref, w_ref, b_ref, y_ref):
    dis = dis_ref[:, :1]
    h = jnp.maximum((a0_ref[...] + a1_ref[...]) * dis + b_ref[...], 0.0)
    y_ref[...] = jnp.dot(h, w_ref[...],
                         preferred_element_type=jnp.float32) * dis


def _tc_b(a0, a1, dis8, w2, b1):
    return pl.pallas_call(
        _tc_b_body,
        grid=(_GRID,),
        in_specs=[
            pl.BlockSpec((_RB, DIM), lambda i: (i, 0)),
            pl.BlockSpec((_RB, DIM), lambda i: (i, 0)),
            pl.BlockSpec((_RB, 8), lambda i: (i, 0)),
            pl.BlockSpec((DIM, DIM), lambda i: (0, 0)),
            pl.BlockSpec((1, DIM), lambda i: (0, 0)),
        ],
        out_specs=pl.BlockSpec((_RB, DIM), lambda i: (i, 0)),
        out_shape=jax.ShapeDtypeStruct((N_NODES, DIM), jnp.float32),
        compiler_params=_tc_params,
    )(a0, a1, dis8, w2, b1)


def _tc_c_body(a0_ref, a1_ref, dis_ref, b_ref, batch_ref, wl_ref, bl_ref,
               out_ref, sums_ref, cnt_ref):
    i = pl.program_id(0)

    @pl.when(i == 0)
    def _():
        sums_ref[...] = jnp.zeros_like(sums_ref)
        cnt_ref[...] = jnp.zeros_like(cnt_ref)

    dis = dis_ref[:, :1]
    h = jnp.maximum((a0_ref[...] + a1_ref[...]) * dis + b_ref[...], 0.0)
    gids = lax.broadcasted_iota(jnp.int32, (_RB, NUM_GRAPHS), 1)
    oh = (batch_ref[...] == gids).astype(jnp.float32)
    sums_ref[...] += lax.dot_general(
        oh, h, (((0,), (0,)), ((), ())), preferred_element_type=jnp.float32)
    cnt_ref[...] += jnp.sum(oh, axis=0)[:, None]

    @pl.when(i == _GRID - 1)
    def _():
        pooled = sums_ref[...] / jnp.maximum(cnt_ref[...], 1.0)
        out_ref[...] = jnp.dot(pooled, wl_ref[...],
                               preferred_element_type=jnp.float32) + bl_ref[...]


def _tc_c(a0, a1, dis8, b2, batch2d, wl, bl):
    return pl.pallas_call(
        _tc_c_body,
        grid=(_GRID,),
        in_specs=[
            pl.BlockSpec((_RB, DIM), lambda i: (i, 0)),
            pl.BlockSpec((_RB, DIM), lambda i: (i, 0)),
            pl.BlockSpec((_RB, 8), lambda i: (i, 0)),
            pl.BlockSpec((1, DIM), lambda i: (0, 0)),
            pl.BlockSpec((_RB, 1), lambda i: (i, 0)),
            pl.BlockSpec((DIM, 1), lambda i: (0, 0)),
            pl.BlockSpec((1, 1), lambda i: (0, 0)),
        ],
        out_specs=pl.BlockSpec((NUM_GRAPHS, 1), lambda i: (0, 0)),
        out_shape=jax.ShapeDtypeStruct((NUM_GRAPHS, 1), jnp.float32),
        scratch_shapes=[
            pltpu.VMEM((NUM_GRAPHS, DIM), jnp.float32),
            pltpu.VMEM((NUM_GRAPHS, 1), jnp.float32),
        ],
        compiler_params=_tc_params,
    )(a0, a1, dis8, b2, batch2d, wl, bl)


def kernel(x, edge_index, batch, W1, b1, W2, b2, Wl, bl):
    src2d = edge_index[0].astype(jnp.int32).reshape(NW * NCHUNK, CHUNK)
    dst2d = edge_index[1].astype(jnp.int32).reshape(NW * NCHUNK, CHUNK)

    ones8 = jnp.ones((CHUNK, DEGW), jnp.float32)
    zeros8 = jnp.zeros((N_NODES, DEGW), jnp.float32)
    zerosD = jnp.zeros((N_NODES, DIM), jnp.float32)

    _DBG_JNP_SC = False
    if _DBG_JNP_SC:
        srcf = src2d.reshape(-1)
        dstf = dst2d.reshape(-1)
        half = N_EDGES // 2
        dega = jnp.zeros((N_NODES,)).at[dstf[:half]].add(1.0)
        degb = jnp.zeros((N_NODES,)).at[dstf[half:]].add(1.0)
        deg = jnp.stack([jnp.broadcast_to(dega[:, None], (N_NODES, 8)),
                         jnp.broadcast_to(degb[:, None], (N_NODES, 8))])

        def _jnp_agg(y):
            a0 = y.at[dstf[:half]].add(y[srcf[:half]])
            a1 = jnp.zeros_like(y).at[dstf[half:]].add(y[srcf[half:]])
            return jnp.stack([a0, a1])
    deg = _deg_kernel(dst2d, ones8, zeros8)
    y1, dis8 = _tc_a(x, W1, deg[0], deg[1])

    def _agg(y):
        if _DBG_LINEAR_GATHER:
            rows = y[src2d.reshape(-1)].reshape(NW * NCHUNK, CHUNK, DIM)
            return _agg_kernel(y, rows, dst2d, zerosD)
        return _agg_kernel(y, src2d, dst2d, zerosD)

    acc1 = _agg(y1) if not _DBG_JNP_SC else _jnp_agg(y1)
    y2 = _tc_b(acc1[0], acc1[1], dis8, W2, b1.reshape(1, DIM))

    acc2 = _agg(y2) if not _DBG_JNP_SC else _jnp_agg(y2)
    out = _tc_c(acc2[0], acc2[1], dis8, b2.reshape(1, DIM),
                batch.astype(jnp.int32).reshape(N_NODES, 1),
                Wl, bl.reshape(1, 1))
    return out


# speedup vs baseline: 25.7091x; 1.3319x over previous
import functools

import jax
import jax.numpy as jnp
from jax import lax
from jax.experimental import pallas as pl
from jax.experimental.pallas import tpu as pltpu
from jax.experimental.pallas import tpu_sc as plsc

N_NODES = 10000
N_EDGES = 320000
DIM = 128
NUM_GRAPHS = 64

NC = 2
NS = 16
NW = NC * NS
EDGES_PER_TILE = N_EDGES // NW
CHUNK = 125
NCHUNK = EDGES_PER_TILE // CHUNK
RPT = 624
TAIL_BASE = NS * RPT
TAIL = N_NODES - TAIL_BASE

_mesh = plsc.VectorSubcoreMesh(core_axis_name="c", subcore_axis_name="s")


DEGW = 128
DEGS = DEGW


@functools.partial(
    pl.kernel,
    mesh=_mesh,
    out_type=jax.ShapeDtypeStruct((NC, N_NODES, DEGW), jnp.float32),
    scratch_types=[
        pltpu.VMEM((NCHUNK, CHUNK), jnp.int32),
        pltpu.VMEM((CHUNK, DEGS), jnp.float32),
        pltpu.VMEM_SHARED((N_NODES, DEGS), jnp.float32),
    ],
)
def _deg_kernel(dst_hbm, ones_hbm, zeros_hbm, out_hbm, dst_v, ones_v, deg_sh):
    cid = lax.axis_index("c")
    sid = lax.axis_index("s")
    wid = cid * NS + sid
    rbase = sid * RPT

    pltpu.sync_copy(dst_hbm.at[pl.ds(wid * NCHUNK, NCHUNK)], dst_v)
    pltpu.sync_copy(ones_hbm, ones_v)
    pltpu.sync_copy(zeros_hbm.at[pl.ds(rbase, RPT)],
                    deg_sh.at[pl.ds(rbase, RPT)])

    @pl.when(sid == NS - 1)
    def _():
        pltpu.sync_copy(zeros_hbm.at[pl.ds(TAIL_BASE, TAIL)],
                        deg_sh.at[pl.ds(TAIL_BASE, TAIL)])

    plsc.subcore_barrier()

    def body(j, carry):
        pltpu.sync_copy(ones_v, deg_sh.at[dst_v.at[j]], add=True)
        return carry

    lax.fori_loop(0, NCHUNK, body, 0)
    plsc.subcore_barrier()
    pltpu.sync_copy(deg_sh.at[pl.ds(rbase, RPT)],
                    out_hbm.at[cid].at[pl.ds(rbase, RPT)])

    @pl.when(sid == NS - 1)
    def _():
        pltpu.sync_copy(deg_sh.at[pl.ds(TAIL_BASE, TAIL)],
                        out_hbm.at[cid].at[pl.ds(TAIL_BASE, TAIL)])


@functools.partial(
    pl.kernel,
    mesh=_mesh,
    out_type=jax.ShapeDtypeStruct((NC, N_NODES, DIM), jnp.float32),
    scratch_types=[
        pltpu.VMEM((NCHUNK // 2, CHUNK), jnp.int32),
        pltpu.VMEM((NCHUNK // 2, CHUNK), jnp.int32),
        pltpu.VMEM((CHUNK, DIM), jnp.float32),
        pltpu.VMEM((CHUNK, DIM), jnp.float32),
        pltpu.VMEM_SHARED((N_NODES, DIM), jnp.float32),
        pltpu.SemaphoreType.DMA,
        pltpu.SemaphoreType.DMA,
    ],
)
def _agg_kernel(y_hbm, src_hbm, dst_hbm, zeros_hbm, out_hbm,
                src_v, dst_v, rows_a, rows_b, acc_sh, sem_a, sem_b):
    cid = lax.axis_index("c")
    sid = lax.axis_index("s")
    wid = cid * NS + sid
    rbase = sid * RPT

    @pl.when(cid == 0)
    def _():
        pltpu.sync_copy(y_hbm.at[pl.ds(rbase, RPT)],
                        acc_sh.at[pl.ds(rbase, RPT)])

        @pl.when(sid == NS - 1)
        def _():
            pltpu.sync_copy(y_hbm.at[pl.ds(TAIL_BASE, TAIL)],
                            acc_sh.at[pl.ds(TAIL_BASE, TAIL)])

    @pl.when(cid == 1)
    def _():
        pltpu.sync_copy(zeros_hbm.at[pl.ds(rbase, RPT)],
                        acc_sh.at[pl.ds(rbase, RPT)])

        @pl.when(sid == NS - 1)
        def _():
            pltpu.sync_copy(zeros_hbm.at[pl.ds(TAIL_BASE, TAIL)],
                            acc_sh.at[pl.ds(TAIL_BASE, TAIL)])

    plsc.subcore_barrier()

    NH = NCHUNK // 2

    def fire(j, buf, sem):
        pltpu.async_copy(y_hbm.at[src_v.at[j]], buf, sem)

    def drain(j, buf, sem):
        pltpu.make_async_copy(y_hbm.at[src_v.at[j]], buf, sem).wait()

    for h in range(2):
        pltpu.sync_copy(src_hbm.at[pl.ds(wid * NCHUNK + h * NH, NH)], src_v)
        pltpu.sync_copy(dst_hbm.at[pl.ds(wid * NCHUNK + h * NH, NH)], dst_v)
        fire(0, rows_a, sem_a)

        def body(i, carry):
            j = 2 * i
            fire(j + 1, rows_b, sem_b)
            drain(j, rows_a, sem_a)
            pltpu.sync_copy(rows_a, acc_sh.at[dst_v.at[j]], add=True)
            fire(jnp.minimum(j + 2, NH - 1), rows_a, sem_a)
            drain(j + 1, rows_b, sem_b)
            pltpu.sync_copy(rows_b, acc_sh.at[dst_v.at[j + 1]], add=True)
            return carry

        lax.fori_loop(0, NH // 2, body, 0)
        drain(NH - 1, rows_a, sem_a)

    plsc.subcore_barrier()
    pltpu.sync_copy(acc_sh.at[pl.ds(rbase, RPT)],
                    out_hbm.at[cid].at[pl.ds(rbase, RPT)])

    @pl.when(sid == NS - 1)
    def _():
        pltpu.sync_copy(acc_sh.at[pl.ds(TAIL_BASE, TAIL)],
                        out_hbm.at[cid].at[pl.ds(TAIL_BASE, TAIL)])


_RB = 1000
_GRID = N_NODES // _RB

_tc_params = pltpu.CompilerParams(dimension_semantics=("arbitrary",))


def _tc_a_body(x_ref, w_ref, da_ref, db_ref, y_ref, dis_ref):
    d = da_ref[:, :1] + db_ref[:, :1] + 1.0
    dis = lax.rsqrt(d)
    xw = jnp.dot(x_ref[...], w_ref[...], preferred_element_type=jnp.float32)
    y_ref[...] = xw * dis
    dis_ref[...] = jnp.broadcast_to(dis, (_RB, 8))


def _tc_a(x, w1, deg_a, deg_b):
    return pl.pallas_call(
        _tc_a_body,
        grid=(_GRID,),
        in_specs=[
            pl.BlockSpec((_RB, DIM), lambda i: (i, 0)),
            pl.BlockSpec((DIM, DIM), lambda i: (0, 0)),
            pl.BlockSpec((_RB, DEGW), lambda i: (i, 0)),
            pl.BlockSpec((_RB, DEGW), lambda i: (i, 0)),
        ],
        out_specs=[
            pl.BlockSpec((_RB, DIM), lambda i: (i, 0)),
            pl.BlockSpec((_RB, 8), lambda i: (i, 0)),
        ],
        out_shape=[
            jax.ShapeDtypeStruct((N_NODES, DIM), jnp.float32),
            jax.ShapeDtypeStruct((N_NODES, 8), jnp.float32),
        ],
        compiler_params=_tc_params,
    )(x, w1, deg_a, deg_b)


def _tc_b_body(a0_ref, a1_ref, dis_ref, w_ref, b_ref, y_ref):
    dis = dis_ref[:, :1]
    h = jnp.maximum((a0_ref[...] + a1_ref[...]) * dis + b_ref[...], 0.0)
    y_ref[...] = jnp.dot(h, w_ref[...],
                         preferred_element_type=jnp.float32) * dis


def _tc_b(a0, a1, dis8, w2, b1):
    return pl.pallas_call(
        _tc_b_body,
        grid=(_GRID,),
        in_specs=[
            pl.BlockSpec((_RB, DIM), lambda i: (i, 0)),
            pl.BlockSpec((_RB, DIM), lambda i: (i, 0)),
            pl.BlockSpec((_RB, 8), lambda i: (i, 0)),
            pl.BlockSpec((DIM, DIM), lambda i: (0, 0)),
            pl.BlockSpec((1, DIM), lambda i: (0, 0)),
        ],
        out_specs=pl.BlockSpec((_RB, DIM), lambda i: (i, 0)),
        out_shape=jax.ShapeDtypeStruct((N_NODES, DIM), jnp.float32),
        compiler_params=_tc_params,
    )(a0, a1, dis8, w2, b1)


def _tc_c_body(a0_ref, a1_ref, dis_ref, b_ref, batch_ref, wl_ref, bl_ref,
               out_ref, sums_ref, cnt_ref):
    i = pl.program_id(0)

    @pl.when(i == 0)
    def _():
        sums_ref[...] = jnp.zeros_like(sums_ref)
        cnt_ref[...] = jnp.zeros_like(cnt_ref)

    dis = dis_ref[:, :1]
    h = jnp.maximum((a0_ref[...] + a1_ref[...]) * dis + b_ref[...], 0.0)
    gids = lax.broadcasted_iota(jnp.int32, (_RB, NUM_GRAPHS), 1)
    oh = (batch_ref[...] == gids).astype(jnp.float32)
    sums_ref[...] += lax.dot_general(
        oh, h, (((0,), (0,)), ((), ())), preferred_element_type=jnp.float32)
    cnt_ref[...] += jnp.sum(oh, axis=0)[:, None]

    @pl.when(i == _GRID - 1)
    def _():
        pooled = sums_ref[...] / jnp.maximum(cnt_ref[...], 1.0)
        out_ref[...] = jnp.dot(pooled, wl_ref[...],
                               preferred_element_type=jnp.float32) + bl_ref[...]


def _tc_c(a0, a1, dis8, b2, batch2d, wl, bl):
    return pl.pallas_call(
        _tc_c_body,
        grid=(_GRID,),
        in_specs=[
            pl.BlockSpec((_RB, DIM), lambda i: (i, 0)),
            pl.BlockSpec((_RB, DIM), lambda i: (i, 0)),
            pl.BlockSpec((_RB, 8), lambda i: (i, 0)),
            pl.BlockSpec((1, DIM), lambda i: (0, 0)),
            pl.BlockSpec((_RB, 1), lambda i: (i, 0)),
            pl.BlockSpec((DIM, 1), lambda i: (0, 0)),
            pl.BlockSpec((1, 1), lambda i: (0, 0)),
        ],
        out_specs=pl.BlockSpec((NUM_GRAPHS, 1), lambda i: (0, 0)),
        out_shape=jax.ShapeDtypeStruct((NUM_GRAPHS, 1), jnp.float32),
        scratch_shapes=[
            pltpu.VMEM((NUM_GRAPHS, DIM), jnp.float32),
            pltpu.VMEM((NUM_GRAPHS, 1), jnp.float32),
        ],
        compiler_params=_tc_params,
    )(a0, a1, dis8, b2, batch2d, wl, bl)


def kernel(x, edge_index, batch, W1, b1, W2, b2, Wl, bl):
    src2d = edge_index[0].astype(jnp.int32).reshape(NW * NCHUNK, CHUNK)
    dst2d = edge_index[1].astype(jnp.int32).reshape(NW * NCHUNK, CHUNK)

    ones8 = jnp.ones((CHUNK, DEGW), jnp.float32)
    zeros8 = jnp.zeros((N_NODES, DEGW), jnp.float32)
    zerosD = jnp.zeros((N_NODES, DIM), jnp.float32)

    deg = _deg_kernel(dst2d, ones8, zeros8)
    y1, dis8 = _tc_a(x, W1, deg[0], deg[1])

    acc1 = _agg_kernel(y1, src2d, dst2d, zerosD)
    y2 = _tc_b(acc1[0], acc1[1], dis8, W2, b1.reshape(1, DIM))

    acc2 = _agg_kernel(y2, src2d, dst2d, zerosD)
    out = _tc_c(acc2[0], acc2[1], dis8, b2.reshape(1, DIM),
                batch.astype(jnp.int32).reshape(N_NODES, 1),
                Wl, bl.reshape(1, 1))
    return out


# speedup vs baseline: 25.7443x; 1.0014x over previous
import functools

import jax
import jax.numpy as jnp
from jax import lax
from jax.experimental import pallas as pl
from jax.experimental.pallas import tpu as pltpu
from jax.experimental.pallas import tpu_sc as plsc

N_NODES = 10000
N_EDGES = 320000
DIM = 128
NUM_GRAPHS = 64

NC = 2
NS = 16
NW = NC * NS
EDGES_PER_TILE = N_EDGES // NW
CHUNK = 125
NCHUNK = EDGES_PER_TILE // CHUNK
RPT = 624
TAIL_BASE = NS * RPT
TAIL = N_NODES - TAIL_BASE

_mesh = plsc.VectorSubcoreMesh(core_axis_name="c", subcore_axis_name="s")


DEGW = 128


@functools.partial(
    pl.kernel,
    mesh=_mesh,
    out_type=jax.ShapeDtypeStruct((NC, N_NODES, DEGW), jnp.float32),
    scratch_types=[
        pltpu.VMEM((NCHUNK, CHUNK), jnp.int32),
        pltpu.VMEM((CHUNK, DEGW), jnp.float32),
        pltpu.VMEM_SHARED((N_NODES, DEGW), jnp.float32),
    ],
)
def _deg_kernel(dst_hbm, ones_hbm, zeros_hbm, out_hbm, dst_v, ones_v, deg_sh):
    cid = lax.axis_index("c")
    sid = lax.axis_index("s")
    wid = cid * NS + sid
    rbase = sid * RPT

    pltpu.sync_copy(dst_hbm.at[pl.ds(wid * NCHUNK, NCHUNK)], dst_v)
    pltpu.sync_copy(ones_hbm, ones_v)
    pltpu.sync_copy(zeros_hbm.at[pl.ds(rbase, RPT)],
                    deg_sh.at[pl.ds(rbase, RPT)])

    @pl.when(sid == NS - 1)
    def _():
        pltpu.sync_copy(zeros_hbm.at[pl.ds(TAIL_BASE, TAIL)],
                        deg_sh.at[pl.ds(TAIL_BASE, TAIL)])

    plsc.subcore_barrier()

    def body(j, carry):
        pltpu.sync_copy(ones_v, deg_sh.at[dst_v.at[j]], add=True)
        return carry

    lax.fori_loop(0, NCHUNK, body, 0)
    plsc.subcore_barrier()
    pltpu.sync_copy(deg_sh.at[pl.ds(rbase, RPT)],
                    out_hbm.at[cid].at[pl.ds(rbase, RPT)])

    @pl.when(sid == NS - 1)
    def _():
        pltpu.sync_copy(deg_sh.at[pl.ds(TAIL_BASE, TAIL)],
                        out_hbm.at[cid].at[pl.ds(TAIL_BASE, TAIL)])


@functools.partial(
    pl.kernel,
    mesh=_mesh,
    out_type=jax.ShapeDtypeStruct((NC, N_NODES, DIM), jnp.float32),
    scratch_types=[
        pltpu.VMEM((NCHUNK // 2, CHUNK), jnp.int32),
        pltpu.VMEM((NCHUNK // 2, CHUNK), jnp.int32),
        pltpu.VMEM((CHUNK, DIM), jnp.float32),
        pltpu.VMEM((CHUNK, DIM), jnp.float32),
        pltpu.VMEM_SHARED((N_NODES, DIM), jnp.float32),
        pltpu.SemaphoreType.DMA,
        pltpu.SemaphoreType.DMA,
    ],
)
def _agg_kernel(y_hbm, src_hbm, dst_hbm, zeros_hbm, out_hbm,
                src_v, dst_v, rows_a, rows_b, acc_sh, sem_a, sem_b):
    cid = lax.axis_index("c")
    sid = lax.axis_index("s")
    wid = cid * NS + sid
    rbase = sid * RPT

    @pl.when(cid == 0)
    def _():
        pltpu.sync_copy(y_hbm.at[pl.ds(rbase, RPT)],
                        acc_sh.at[pl.ds(rbase, RPT)])

        @pl.when(sid == NS - 1)
        def _():
            pltpu.sync_copy(y_hbm.at[pl.ds(TAIL_BASE, TAIL)],
                            acc_sh.at[pl.ds(TAIL_BASE, TAIL)])

    @pl.when(cid == 1)
    def _():
        pltpu.sync_copy(zeros_hbm.at[pl.ds(rbase, RPT)],
                        acc_sh.at[pl.ds(rbase, RPT)])

        @pl.when(sid == NS - 1)
        def _():
            pltpu.sync_copy(zeros_hbm.at[pl.ds(TAIL_BASE, TAIL)],
                            acc_sh.at[pl.ds(TAIL_BASE, TAIL)])

    plsc.subcore_barrier()

    NH = NCHUNK // 2

    def fire(j, buf, sem):
        pltpu.async_copy(y_hbm.at[src_v.at[j]], buf, sem)

    def drain(j, buf, sem):
        pltpu.make_async_copy(y_hbm.at[src_v.at[j]], buf, sem).wait()

    for h in range(2):
        pltpu.sync_copy(src_hbm.at[pl.ds(wid * NCHUNK + h * NH, NH)], src_v)
        pltpu.sync_copy(dst_hbm.at[pl.ds(wid * NCHUNK + h * NH, NH)], dst_v)
        fire(0, rows_a, sem_a)

        def body(i, carry):
            j = 2 * i
            fire(j + 1, rows_b, sem_b)
            drain(j, rows_a, sem_a)
            pltpu.sync_copy(rows_a, acc_sh.at[dst_v.at[j]], add=True)
            fire(jnp.minimum(j + 2, NH - 1), rows_a, sem_a)
            drain(j + 1, rows_b, sem_b)
            pltpu.sync_copy(rows_b, acc_sh.at[dst_v.at[j + 1]], add=True)
            return carry

        lax.fori_loop(0, NH // 2, body, 0)
        drain(NH - 1, rows_a, sem_a)

    plsc.subcore_barrier()
    pltpu.sync_copy(acc_sh.at[pl.ds(rbase, RPT)],
                    out_hbm.at[cid].at[pl.ds(rbase, RPT)])

    @pl.when(sid == NS - 1)
    def _():
        pltpu.sync_copy(acc_sh.at[pl.ds(TAIL_BASE, TAIL)],
                        out_hbm.at[cid].at[pl.ds(TAIL_BASE, TAIL)])


_RB = 1000
_GRID = N_NODES // _RB

_tc_params = pltpu.CompilerParams(dimension_semantics=("arbitrary",))


def _tc_a_body(x_ref, w_ref, da_ref, db_ref, y_ref, dis_ref):
    d = da_ref[:, :1] + db_ref[:, :1] + 1.0
    dis = lax.rsqrt(d)
    xw = jnp.dot(x_ref[...], w_ref[...], preferred_element_type=jnp.float32)
    y_ref[...] = xw * dis
    dis_ref[...] = jnp.broadcast_to(dis, (_RB, 8))


def _tc_a(x, w1, deg_a, deg_b):
    return pl.pallas_call(
        _tc_a_body,
        grid=(_GRID,),
        in_specs=[
            pl.BlockSpec((_RB, DIM), lambda i: (i, 0)),
            pl.BlockSpec((DIM, DIM), lambda i: (0, 0)),
            pl.BlockSpec((_RB, DEGW), lambda i: (i, 0)),
            pl.BlockSpec((_RB, DEGW), lambda i: (i, 0)),
        ],
        out_specs=[
            pl.BlockSpec((_RB, DIM), lambda i: (i, 0)),
            pl.BlockSpec((_RB, 8), lambda i: (i, 0)),
        ],
        out_shape=[
            jax.ShapeDtypeStruct((N_NODES, DIM), jnp.float32),
            jax.ShapeDtypeStruct((N_NODES, 8), jnp.float32),
        ],
        compiler_params=_tc_params,
    )(x, w1, deg_a, deg_b)


def _tc_b_body(a0_ref, a1_ref, dis_ref, w_ref, b_ref, y_ref):
    dis = dis_ref[:, :1]
    h = jnp.maximum((a0_ref[...] + a1_ref[...]) * dis + b_ref[...], 0.0)
    y_ref[...] = jnp.dot(h, w_ref[...],
                         preferred_element_type=jnp.float32) * dis


def _tc_b(a0, a1, dis8, w2, b1):
    return pl.pallas_call(
        _tc_b_body,
        grid=(_GRID,),
        in_specs=[
            pl.BlockSpec((_RB, DIM), lambda i: (i, 0)),
            pl.BlockSpec((_RB, DIM), lambda i: (i, 0)),
            pl.BlockSpec((_RB, 8), lambda i: (i, 0)),
            pl.BlockSpec((DIM, DIM), lambda i: (0, 0)),
            pl.BlockSpec((1, DIM), lambda i: (0, 0)),
        ],
        out_specs=pl.BlockSpec((_RB, DIM), lambda i: (i, 0)),
        out_shape=jax.ShapeDtypeStruct((N_NODES, DIM), jnp.float32),
        compiler_params=_tc_params,
    )(a0, a1, dis8, w2, b1)


def _tc_c_body(a0_ref, a1_ref, dis_ref, b_ref, batch_ref, wl_ref, bl_ref,
               out_ref, sums_ref, cnt_ref):
    i = pl.program_id(0)

    @pl.when(i == 0)
    def _():
        sums_ref[...] = jnp.zeros_like(sums_ref)
        cnt_ref[...] = jnp.zeros_like(cnt_ref)

    dis = dis_ref[:, :1]
    h = jnp.maximum((a0_ref[...] + a1_ref[...]) * dis + b_ref[...], 0.0)
    gids = lax.broadcasted_iota(jnp.int32, (_RB, NUM_GRAPHS), 1)
    oh = (batch_ref[...] == gids).astype(jnp.float32)
    sums_ref[...] += lax.dot_general(
        oh, h, (((0,), (0,)), ((), ())), preferred_element_type=jnp.float32)
    cnt_ref[...] += jnp.sum(oh, axis=0)[:, None]

    @pl.when(i == _GRID - 1)
    def _():
        pooled = sums_ref[...] / jnp.maximum(cnt_ref[...], 1.0)
        out_ref[...] = jnp.dot(pooled, wl_ref[...],
                               preferred_element_type=jnp.float32) + bl_ref[...]


def _tc_c(a0, a1, dis8, b2, batch2d, wl, bl):
    return pl.pallas_call(
        _tc_c_body,
        grid=(_GRID,),
        in_specs=[
            pl.BlockSpec((_RB, DIM), lambda i: (i, 0)),
            pl.BlockSpec((_RB, DIM), lambda i: (i, 0)),
            pl.BlockSpec((_RB, 8), lambda i: (i, 0)),
            pl.BlockSpec((1, DIM), lambda i: (0, 0)),
            pl.BlockSpec((_RB, 1), lambda i: (i, 0)),
            pl.BlockSpec((DIM, 1), lambda i: (0, 0)),
            pl.BlockSpec((1, 1), lambda i: (0, 0)),
        ],
        out_specs=pl.BlockSpec((NUM_GRAPHS, 1), lambda i: (0, 0)),
        out_shape=jax.ShapeDtypeStruct((NUM_GRAPHS, 1), jnp.float32),
        scratch_shapes=[
            pltpu.VMEM((NUM_GRAPHS, DIM), jnp.float32),
            pltpu.VMEM((NUM_GRAPHS, 1), jnp.float32),
        ],
        compiler_params=_tc_params,
    )(a0, a1, dis8, b2, batch2d, wl, bl)


def kernel(x, edge_index, batch, W1, b1, W2, b2, Wl, bl):
    src2d = edge_index[0].astype(jnp.int32).reshape(NW * NCHUNK, CHUNK)
    dst2d = edge_index[1].astype(jnp.int32).reshape(NW * NCHUNK, CHUNK)

    zerosD = jnp.zeros((N_NODES, DIM), jnp.float32)
    onesW = jnp.ones((CHUNK, DEGW), jnp.float32)
    zerosW = jnp.zeros((N_NODES, DEGW), jnp.float32)

    deg = _deg_kernel(dst2d, onesW, zerosW)
    y1, dis8 = _tc_a(x, W1, deg[0], deg[1])

    acc1 = _agg_kernel(y1, src2d, dst2d, zerosD)
    y2 = _tc_b(acc1[0], acc1[1], dis8, W2, b1.reshape(1, DIM))

    acc2 = _agg_kernel(y2, src2d, dst2d, zerosD)
    out = _tc_c(acc2[0], acc2[1], dis8, b2.reshape(1, DIM),
                batch.astype(jnp.int32).reshape(N_NODES, 1),
                Wl, bl.reshape(1, 1))
    return out
